# baseline (device time: 10023 ns/iter reference)
import jax
import jax.numpy as jnp
from jax import lax
from jax.experimental import pallas as pl
from jax.experimental.pallas import tpu as pltpu

BLOCK_M = 256


def kernel(x):
    m, n = x.shape
    m_global = 2 * m
    num = m // BLOCK_M

    def chunk_copy(x_hbm, buf, copy_sems, i):
        return pltpu.make_async_copy(
            x_hbm.at[pl.ds(i * BLOCK_M, BLOCK_M), :],
            buf.at[i % 2],
            copy_sems.at[i % 2],
        )

    def body(x_hbm, out_hbm, buf, res, comm_ref, copy_sems, out_sem,
             send_sem, recv_sem):
        my_x = lax.axis_index("x")
        my_y = lax.axis_index("y")
        peer = (1 - my_x, my_y)

        barrier_sem = pltpu.get_barrier_semaphore()
        pl.semaphore_signal(
            barrier_sem, inc=1, device_id=peer,
            device_id_type=pl.DeviceIdType.MESH,
        )

        chunk_copy(x_hbm, buf, copy_sems, 0).start()
        chunk_copy(x_hbm, buf, copy_sems, 1).start()

        acc = None
        for i in range(num):
            chunk_copy(x_hbm, buf, copy_sems, i).wait()
            part = jnp.sum(buf[i % 2], axis=0, keepdims=True)
            acc = part if acc is None else acc + part
            if i + 2 < num:
                chunk_copy(x_hbm, buf, copy_sems, i + 2).start()

        comm_ref[0, :, :] = acc
        pl.semaphore_wait(barrier_sem, 1)

        rdma = pltpu.make_async_remote_copy(
            src_ref=comm_ref.at[0],
            dst_ref=comm_ref.at[1],
            send_sem=send_sem,
            recv_sem=recv_sem,
            device_id=peer,
            device_id_type=pl.DeviceIdType.MESH,
        )
        rdma.start()
        rdma.wait()

        res[:, :] = (comm_ref[0, :, :] + comm_ref[1, :, :]) * (1.0 / m_global)
        out_cp = pltpu.make_async_copy(res, out_hbm, out_sem)
        out_cp.start()
        out_cp.wait()

    x = pltpu.with_memory_space_constraint(x, pltpu.MemorySpace.HBM)
    return pl.pallas_call(
        body,
        out_shape=jax.ShapeDtypeStruct((1, n), jnp.float32),
        in_specs=[pl.BlockSpec(memory_space=pltpu.MemorySpace.HBM)],
        out_specs=pl.BlockSpec(memory_space=pltpu.MemorySpace.HBM),
        scratch_shapes=[
            pltpu.VMEM((2, BLOCK_M, n), jnp.float32),
            pltpu.VMEM((1, n), jnp.float32),
            pltpu.VMEM((2, 1, n), jnp.float32),
            pltpu.SemaphoreType.DMA((2,)),
            pltpu.SemaphoreType.DMA,
            pltpu.SemaphoreType.DMA,
            pltpu.SemaphoreType.DMA,
        ],
        compiler_params=pltpu.CompilerParams(collective_id=0),
    )(x)


# device time: 7857 ns/iter; 1.2757x vs baseline; 1.2757x over previous
import jax
import jax.numpy as jnp
from jax import lax
from jax.experimental import pallas as pl
from jax.experimental.pallas import tpu as pltpu

NUM_CHUNKS = 8


def kernel(x):
    m, n = x.shape
    m_global = 2 * m
    block_m = m // NUM_CHUNKS

    def chunk_copy(x_hbm, buf, copy_sems, i):
        return pltpu.make_async_copy(
            x_hbm.at[pl.ds(i * block_m, block_m), :],
            buf.at[i],
            copy_sems.at[i],
        )

    def body(x_hbm, out_ref, buf, comm_ref, copy_sems, send_sem, recv_sem):
        my_x = lax.axis_index("x")
        my_y = lax.axis_index("y")
        peer = (1 - my_x, my_y)

        barrier_sem = pltpu.get_barrier_semaphore()
        pl.semaphore_signal(
            barrier_sem, inc=1, device_id=peer,
            device_id_type=pl.DeviceIdType.MESH,
        )

        for i in range(NUM_CHUNKS):
            chunk_copy(x_hbm, buf, copy_sems, i).start()

        acc = None
        for i in range(NUM_CHUNKS):
            chunk_copy(x_hbm, buf, copy_sems, i).wait()
            part = jnp.sum(buf[i], axis=0, keepdims=True)
            acc = part if acc is None else acc + part

        comm_ref[0, :, :] = acc
        pl.semaphore_wait(barrier_sem, 1)

        rdma = pltpu.make_async_remote_copy(
            src_ref=comm_ref.at[0],
            dst_ref=comm_ref.at[1],
            send_sem=send_sem,
            recv_sem=recv_sem,
            device_id=peer,
            device_id_type=pl.DeviceIdType.MESH,
        )
        rdma.start()
        rdma.wait()

        out_ref[:, :] = (comm_ref[0, :, :] + comm_ref[1, :, :]) * (
            1.0 / m_global
        )

    x = pltpu.with_memory_space_constraint(x, pltpu.MemorySpace.HBM)
    return pl.pallas_call(
        body,
        out_shape=jax.ShapeDtypeStruct((1, n), jnp.float32),
        in_specs=[pl.BlockSpec(memory_space=pltpu.MemorySpace.HBM)],
        out_specs=pl.BlockSpec(memory_space=pltpu.VMEM),
        scratch_shapes=[
            pltpu.VMEM((NUM_CHUNKS, block_m, n), jnp.float32),
            pltpu.VMEM((2, 1, n), jnp.float32),
            pltpu.SemaphoreType.DMA((NUM_CHUNKS,)),
            pltpu.SemaphoreType.DMA,
            pltpu.SemaphoreType.DMA,
        ],
        compiler_params=pltpu.CompilerParams(collective_id=0),
    )(x)
